# baseline (speedup 1.0000x reference)
import functools

import jax
import jax.numpy as jnp
from jax import lax
from jax.experimental import pallas as pl
from jax.experimental.pallas import tpu as pltpu
from jax.experimental.pallas import tpu_sc as plsc

NUM_MODELS = 1000000
NUM_TASKS = 100000
D = 32
B = 16384
L = 16
NW = 32
BPW = B // NW
CHUNK = 128
NCHUNK = BPW // CHUNK


def _make_sc_kernel():
    mesh = plsc.VectorSubcoreMesh(core_axis_name="c", subcore_axis_name="s")

    @functools.partial(
        pl.kernel,
        mesh=mesh,
        out_type=jax.ShapeDtypeStruct((B,), jnp.float32),
        compiler_params=pltpu.CompilerParams(
            needs_layout_passes=False, use_tc_tiling_on_sc=False),
        scratch_types=[
            pltpu.VMEM((NCHUNK, CHUNK), jnp.int32),
            pltpu.VMEM((NCHUNK, CHUNK), jnp.int32),
            pltpu.VMEM((BPW, D), jnp.float32),
            pltpu.VMEM((BPW, D), jnp.float32),
            pltpu.VMEM((BPW,), jnp.float32),
            pltpu.SemaphoreType.DMA,
        ],
    )
    def k(model_hbm, task_hbm, mf_hbm, tf_hbm, out_hbm,
          idx_m, idx_t, mrows, trows, out_v, sem):
        wid = lax.axis_index("s") * 2 + lax.axis_index("c")
        base = wid * BPW

        for j in range(NCHUNK):
            pltpu.sync_copy(model_hbm.at[pl.ds(base + j * CHUNK, CHUNK)],
                            idx_m.at[j])
            pltpu.sync_copy(task_hbm.at[pl.ds(base + j * CHUNK, CHUNK)],
                            idx_t.at[j])

        copies = []
        for j in range(NCHUNK):
            copies.append(pltpu.async_copy(
                mf_hbm.at[idx_m.at[j]],
                mrows.at[pl.ds(j * CHUNK, CHUNK)], sem))
            copies.append(pltpu.async_copy(
                tf_hbm.at[idx_t.at[j]],
                trows.at[pl.ds(j * CHUNK, CHUNK)], sem))
        for c in copies:
            c.wait()

        iota = lax.iota(jnp.int32, L)

        def group_body(g, carry):
            row = pl.multiple_of(g * L, L) + iota
            acc = jnp.zeros((L,), jnp.float32)
            for d in range(D):
                col = jnp.full((L,), d, jnp.int32)
                mv = plsc.load_gather(mrows, [row, col])
                tv = plsc.load_gather(trows, [row, col])
                acc = acc + mv * tv
            out_v[pl.ds(pl.multiple_of(g * L, L), L)] = acc
            return carry

        lax.fori_loop(0, BPW // L, group_body, 0)

        pltpu.sync_copy(out_v, out_hbm.at[pl.ds(base, BPW)])

    return k


_sc_kernel = _make_sc_kernel()


def kernel(model, task, model_factors, task_factors):
    model = model.astype(jnp.int32)
    task = task.astype(jnp.int32)
    return _sc_kernel(model, task, model_factors, task_factors)


# speedup vs baseline: 2.9189x; 2.9189x over previous
import functools

import jax
import jax.numpy as jnp
from jax import lax
from jax.experimental import pallas as pl
from jax.experimental.pallas import tpu as pltpu
from jax.experimental.pallas import tpu_sc as plsc

D = 32
B = 16384
L = 16
NW = 32
BPW = B // NW
KCH = 16
ICH = 128


def _make_k1():
    mesh = plsc.VectorSubcoreMesh(core_axis_name="c", subcore_axis_name="s")

    @functools.partial(
        pl.kernel,
        mesh=mesh,
        out_type=jax.ShapeDtypeStruct((B * D,), jnp.float32),
        compiler_params=pltpu.CompilerParams(
            needs_layout_passes=False, use_tc_tiling_on_sc=True),
        scratch_types=[
            pltpu.VMEM((BPW,), jnp.int32),
            pltpu.VMEM((KCH, D, 128), jnp.float32),
            pltpu.VMEM((BPW * D,), jnp.float32),
            pltpu.SemaphoreType.DMA,
        ],
    )
    def k1(model_hbm, mt_hbm, mout_hbm, idx_m, cols, rows, sem):
        wid = lax.axis_index("s") * 2 + lax.axis_index("c")
        base = wid * BPW

        pltpu.sync_copy(model_hbm.at[pl.ds(base, BPW)], idx_m)
        iota = lax.iota(jnp.int32, L)

        def chunk_body(ci, carry):
            c0 = pl.multiple_of(ci * KCH, KCH)
            idx_vec = idx_m[pl.ds(c0, KCH)]
            col_base = (idx_vec >> 7) << 7
            copies = []
            for i in range(KCH):
                cb = pl.multiple_of(col_base[i], 128)
                copies.append(pltpu.async_copy(
                    mt_hbm.at[:, pl.ds(cb, 128)], cols.at[i], sem))
            for cpy in copies:
                cpy.wait()
            lane = idx_vec & 127
            for i in range(KCH):
                lane_col = jnp.full((L,), lane[i], jnp.int32)
                v0 = plsc.load_gather(cols.at[i], [iota, lane_col])
                v1 = plsc.load_gather(cols.at[i], [iota + L, lane_col])
                r0 = pl.multiple_of((c0 + i) * D, D)
                rows[pl.ds(r0, L)] = v0
                rows[pl.ds(r0 + L, L)] = v1
            return carry

        lax.fori_loop(0, BPW // KCH, chunk_body, 0)
        pltpu.sync_copy(rows, mout_hbm.at[pl.ds(base * D, BPW * D)])

    return k1


def _make_k2():
    mesh = plsc.VectorSubcoreMesh(core_axis_name="c", subcore_axis_name="s")

    @functools.partial(
        pl.kernel,
        mesh=mesh,
        out_type=jax.ShapeDtypeStruct((B,), jnp.float32),
        compiler_params=pltpu.CompilerParams(
            needs_layout_passes=False, use_tc_tiling_on_sc=False),
        scratch_types=[
            pltpu.VMEM((BPW // ICH, ICH), jnp.int32),
            pltpu.VMEM((BPW, D), jnp.float32),
            pltpu.VMEM((BPW * D,), jnp.float32),
            pltpu.VMEM((BPW,), jnp.float32),
            pltpu.SemaphoreType.DMA,
        ],
    )
    def k2(task_hbm, tf_hbm, mvec_hbm, out_hbm,
           idx_t, trows, mrows, out_v, sem):
        wid = lax.axis_index("s") * 2 + lax.axis_index("c")
        base = wid * BPW

        for j in range(BPW // ICH):
            pltpu.sync_copy(task_hbm.at[pl.ds(base + j * ICH, ICH)],
                            idx_t.at[j])
        copies = [pltpu.async_copy(
            mvec_hbm.at[pl.ds(base * D, BPW * D)], mrows, sem)]
        for j in range(BPW // ICH):
            copies.append(pltpu.async_copy(
                tf_hbm.at[idx_t.at[j]],
                trows.at[pl.ds(j * ICH, ICH)], sem))
        for cpy in copies:
            cpy.wait()

        iota = lax.iota(jnp.int32, L)

        def group_body(g, carry):
            row = pl.multiple_of(g * L, L) + iota
            flat = row * D
            acc = jnp.zeros((L,), jnp.float32)
            for d in range(D):
                col = jnp.full((L,), d, jnp.int32)
                mv = plsc.load_gather(mrows, [flat + d])
                tv = plsc.load_gather(trows, [row, col])
                acc = acc + mv * tv
            out_v[pl.ds(pl.multiple_of(g * L, L), L)] = acc
            return carry

        lax.fori_loop(0, BPW // L, group_body, 0)
        pltpu.sync_copy(out_v, out_hbm.at[pl.ds(base, BPW)])

    return k2


_k1 = _make_k1()
_k2 = _make_k2()


def kernel(model, task, model_factors, task_factors):
    model = model.astype(jnp.int32)
    task = task.astype(jnp.int32)
    mvec = _k1(model, model_factors.T)
    return _k2(task, task_factors, mvec)


# speedup vs baseline: 2.9690x; 1.0172x over previous
import functools

import jax
import jax.numpy as jnp
from jax import lax
from jax.experimental import pallas as pl
from jax.experimental.pallas import tpu as pltpu
from jax.experimental.pallas import tpu_sc as plsc

D = 32
B = 16384
L = 16
NW = 32
BPW = B // NW
KCH = 8
NKCH = BPW // KCH
ICH = 128


def _make_k1():
    mesh = plsc.VectorSubcoreMesh(core_axis_name="c", subcore_axis_name="s")

    @functools.partial(
        pl.kernel,
        mesh=mesh,
        out_type=jax.ShapeDtypeStruct((B * D,), jnp.float32),
        compiler_params=pltpu.CompilerParams(
            needs_layout_passes=False, use_tc_tiling_on_sc=True),
        scratch_types=[
            pltpu.VMEM((BPW,), jnp.int32),
            pltpu.VMEM((2, KCH, D, 128), jnp.float32),
            pltpu.VMEM((BPW * D,), jnp.float32),
            pltpu.SemaphoreType.DMA,
            pltpu.SemaphoreType.DMA,
        ],
    )
    def k1(model_hbm, mt_hbm, mout_hbm, idx_m, cols, rows, sem0, sem1):
        wid = lax.axis_index("s") * 2 + lax.axis_index("c")
        base = wid * BPW

        pltpu.sync_copy(model_hbm.at[pl.ds(base, BPW)], idx_m)
        iota = lax.iota(jnp.int32, L)
        sems = (sem0, sem1)

        def load_idx(p):
            return idx_m[pl.ds(pl.multiple_of(p * 2 * KCH, 2 * KCH), L)]

        def fire(col_base, parity):
            for i in range(KCH):
                cb = pl.multiple_of(col_base[parity * KCH + i], 128)
                pltpu.async_copy(
                    mt_hbm.at[:, pl.ds(cb, 128)],
                    cols.at[parity, i], sems[parity])

        def drain_extract(p, lane, parity):
            for i in range(KCH):
                pltpu.make_async_copy(
                    mt_hbm.at[:, pl.ds(0, 128)],
                    cols.at[parity, i], sems[parity]).wait()
            c0 = pl.multiple_of((p * 2 + parity) * KCH, KCH)
            for i in range(KCH):
                lane_col = jnp.full((L,), lane[parity * KCH + i], jnp.int32)
                v0 = plsc.load_gather(cols.at[parity, i], [iota, lane_col])
                v1 = plsc.load_gather(cols.at[parity, i], [iota + L, lane_col])
                r0 = pl.multiple_of((c0 + i) * D, D)
                rows[pl.ds(r0, L)] = v0
                rows[pl.ds(r0 + L, L)] = v1

        idx0 = load_idx(0)
        fire((idx0 >> 7) << 7, 0)

        def pair_body(p, carry):
            idx_vec = load_idx(p)
            col_base = (idx_vec >> 7) << 7
            lane = idx_vec & 127
            fire(col_base, 1)
            drain_extract(p, lane, 0)

            @pl.when(p + 1 < NKCH // 2)
            def _():
                idx_nxt = load_idx(p + 1)
                fire((idx_nxt >> 7) << 7, 0)

            drain_extract(p, lane, 1)
            return carry

        lax.fori_loop(0, NKCH // 2, pair_body, 0)
        pltpu.sync_copy(rows, mout_hbm.at[pl.ds(base * D, BPW * D)])

    return k1


def _make_k2():
    mesh = plsc.VectorSubcoreMesh(core_axis_name="c", subcore_axis_name="s")

    @functools.partial(
        pl.kernel,
        mesh=mesh,
        out_type=jax.ShapeDtypeStruct((B,), jnp.float32),
        compiler_params=pltpu.CompilerParams(
            needs_layout_passes=False, use_tc_tiling_on_sc=False),
        scratch_types=[
            pltpu.VMEM((BPW // ICH, ICH), jnp.int32),
            pltpu.VMEM((BPW, D), jnp.float32),
            pltpu.VMEM((BPW * D,), jnp.float32),
            pltpu.VMEM((BPW,), jnp.float32),
            pltpu.SemaphoreType.DMA,
        ],
    )
    def k2(task_hbm, tf_hbm, mvec_hbm, out_hbm,
           idx_t, trows, mrows, out_v, sem):
        wid = lax.axis_index("s") * 2 + lax.axis_index("c")
        base = wid * BPW

        for j in range(BPW // ICH):
            pltpu.sync_copy(task_hbm.at[pl.ds(base + j * ICH, ICH)],
                            idx_t.at[j])
        copies = [pltpu.async_copy(
            mvec_hbm.at[pl.ds(base * D, BPW * D)], mrows, sem)]
        for j in range(BPW // ICH):
            copies.append(pltpu.async_copy(
                tf_hbm.at[idx_t.at[j]],
                trows.at[pl.ds(j * ICH, ICH)], sem))
        for cpy in copies:
            cpy.wait()

        iota = lax.iota(jnp.int32, L)

        def group_body(g, carry):
            row = pl.multiple_of(g * L, L) + iota
            flat = row * D
            acc = jnp.zeros((L,), jnp.float32)
            for d in range(D):
                col = jnp.full((L,), d, jnp.int32)
                mv = plsc.load_gather(mrows, [flat + d])
                tv = plsc.load_gather(trows, [row, col])
                acc = acc + mv * tv
            out_v[pl.ds(pl.multiple_of(g * L, L), L)] = acc
            return carry

        lax.fori_loop(0, BPW // L, group_body, 0)
        pltpu.sync_copy(out_v, out_hbm.at[pl.ds(base, BPW)])

    return k2


_k1 = _make_k1()
_k2 = _make_k2()


def kernel(model, task, model_factors, task_factors):
    model = model.astype(jnp.int32)
    task = task.astype(jnp.int32)
    mvec = _k1(model, model_factors.T)
    return _k2(task, task_factors, mvec)
